# Initial kernel scaffold; baseline (speedup 1.0000x reference)
import jax
import jax.numpy as jnp
from jax.experimental import pallas as pl


def kernel(x, table):
    raise NotImplementedError("write your pallas kernel here")


# speedup vs baseline: 13.5814x; 13.5814x over previous
import functools

import jax
import jax.numpy as jnp
from jax import lax
from jax.experimental import pallas as pl
from jax.experimental.pallas import tpu as pltpu
from jax.experimental.pallas import tpu_sc as plsc

VOCAB = 100000
EMBED = 64
BATCH = 4096
HIST = 200

NC = 2
NS = 16
LANES = 16
NW = NC * NS
BPW = BATCH // NW
HALF = HIST // 2
NVEC = EMBED // LANES

_mesh = plsc.VectorSubcoreMesh(core_axis_name="c", subcore_axis_name="s")


@functools.partial(
    pl.kernel,
    out_type=jax.ShapeDtypeStruct((BATCH, EMBED), jnp.float32),
    mesh=_mesh,
    compiler_params=pltpu.CompilerParams(use_tc_tiling_on_sc=False),
    scratch_types=[
        pltpu.VMEM((2 * BPW, HALF), jnp.int32),
        pltpu.VMEM((2, HIST, EMBED), jnp.float32),
        pltpu.VMEM((BPW, EMBED), jnp.float32),
        pltpu.SemaphoreType.DMA,
        pltpu.SemaphoreType.DMA,
    ],
)
def _embed_pool(x_hbm, table_hbm, dummy_hbm, out_hbm, idx_v, rows_v, out_v,
                sem0, sem1):
    sems = (sem0, sem1)
    wid = lax.axis_index("s") * NC + lax.axis_index("c")
    base = wid * BPW

    pltpu.sync_copy(x_hbm.at[pl.ds(2 * base, 2 * BPW)], idx_v)

    def issue(b, buf):
        pltpu.async_copy(table_hbm.at[idx_v.at[2 * b]],
                         rows_v.at[buf, pl.ds(0, HALF)], sems[buf])
        pltpu.async_copy(table_hbm.at[idx_v.at[2 * b + 1]],
                         rows_v.at[buf, pl.ds(HALF, HALF)], sems[buf])

    def wait(buf):
        pltpu.make_async_copy(dummy_hbm, rows_v.at[buf], sems[buf]).wait()

    def reduce(b, buf):
        def body(l, acc):
            return tuple(acc[i] + rows_v[buf, l, pl.ds(LANES * i, LANES)]
                         for i in range(NVEC))
        acc = lax.fori_loop(
            0, HIST, body,
            tuple(jnp.zeros((LANES,), jnp.float32) for _ in range(NVEC)))
        for i in range(NVEC):
            out_v[b, pl.ds(LANES * i, LANES)] = acc[i] * (1.0 / HIST)

    issue(0, 0)

    def outer(j, _):
        b0 = 2 * j
        issue(b0 + 1, 1)
        wait(0)
        reduce(b0, 0)

        @pl.when(j < BPW // 2 - 1)
        def _():
            issue(b0 + 2, 0)

        wait(1)
        reduce(b0 + 1, 1)
        return 0

    lax.fori_loop(0, BPW // 2, outer, 0)
    pltpu.sync_copy(out_v, out_hbm.at[pl.ds(base, BPW)])


def kernel(x, table):
    x2 = x.astype(jnp.int32).reshape(2 * BATCH, HALF)
    dummy = jnp.zeros((HIST, EMBED), jnp.float32)
    return _embed_pool(x2, table, dummy)


# speedup vs baseline: 14.3975x; 1.0601x over previous
import functools

import jax
import jax.numpy as jnp
from jax import lax
from jax.experimental import pallas as pl
from jax.experimental.pallas import tpu as pltpu
from jax.experimental.pallas import tpu_sc as plsc

VOCAB = 100000
EMBED = 64
BATCH = 4096
HIST = 200

NC = 2
NS = 16
LANES = 16
NW = NC * NS
BPW = BATCH // NW
HALF = HIST // 2
NVEC = EMBED // LANES

_mesh = plsc.VectorSubcoreMesh(core_axis_name="c", subcore_axis_name="s")


@functools.partial(
    pl.kernel,
    out_type=jax.ShapeDtypeStruct((BATCH, EMBED), jnp.float32),
    mesh=_mesh,
    compiler_params=pltpu.CompilerParams(use_tc_tiling_on_sc=False),
    scratch_types=[
        pltpu.VMEM((2 * BPW, HALF), jnp.int32),
        pltpu.VMEM((2, HIST, EMBED), jnp.float32),
        pltpu.VMEM((BPW, EMBED), jnp.float32),
        pltpu.SemaphoreType.DMA,
        pltpu.SemaphoreType.DMA,
    ],
)
def _embed_pool(x_hbm, table_hbm, dummy_hbm, out_hbm, idx_v, rows_v, out_v,
                sem0, sem1):
    sems = (sem0, sem1)
    wid = lax.axis_index("s") * NC + lax.axis_index("c")
    base = wid * BPW

    pltpu.sync_copy(x_hbm.at[pl.ds(2 * base, 2 * BPW)], idx_v)

    def issue(b, buf):
        pltpu.async_copy(table_hbm.at[idx_v.at[2 * b]],
                         rows_v.at[buf, pl.ds(0, HALF)], sems[buf])
        pltpu.async_copy(table_hbm.at[idx_v.at[2 * b + 1]],
                         rows_v.at[buf, pl.ds(HALF, HALF)], sems[buf])

    def wait(buf):
        pltpu.make_async_copy(dummy_hbm, rows_v.at[buf], sems[buf]).wait()

    def reduce(b, buf):
        U = 8

        def body(t, acc):
            acc = list(acc)
            for u in range(U):
                c = NVEC * (u % 2)
                for i in range(NVEC):
                    acc[c + i] = acc[c + i] + rows_v[
                        buf, t * U + u, pl.ds(LANES * i, LANES)]
            return tuple(acc)

        acc = lax.fori_loop(
            0, HIST // U, body,
            tuple(jnp.zeros((LANES,), jnp.float32) for _ in range(2 * NVEC)))
        for i in range(NVEC):
            out_v[b, pl.ds(LANES * i, LANES)] = (
                (acc[i] + acc[NVEC + i]) * (1.0 / HIST))

    issue(0, 0)

    def outer(j, _):
        b0 = 2 * j
        issue(b0 + 1, 1)
        wait(0)
        reduce(b0, 0)

        @pl.when(j < BPW // 2 - 1)
        def _():
            issue(b0 + 2, 0)

        wait(1)
        reduce(b0 + 1, 1)
        return 0

    lax.fori_loop(0, BPW // 2, outer, 0)
    pltpu.sync_copy(out_v, out_hbm.at[pl.ds(base, BPW)])


def kernel(x, table):
    x2 = x.astype(jnp.int32).reshape(2 * BATCH, HALF)
    dummy = jnp.zeros((HIST, EMBED), jnp.float32)
    return _embed_pool(x2, table, dummy)


# speedup vs baseline: 17.8799x; 1.2419x over previous
import functools

import jax
import jax.numpy as jnp
from jax import lax
from jax.experimental import pallas as pl
from jax.experimental.pallas import tpu as pltpu
from jax.experimental.pallas import tpu_sc as plsc

VOCAB = 100000
EMBED = 64
BATCH = 4096
HIST = 200

NC = 2
NS = 16
LANES = 16
NW = NC * NS
BPW = BATCH // NW
HALF = HIST // 2
NVEC = EMBED // LANES

_mesh = plsc.VectorSubcoreMesh(core_axis_name="c", subcore_axis_name="s")


@functools.partial(
    pl.kernel,
    out_type=jax.ShapeDtypeStruct((BATCH, EMBED), jnp.float32),
    mesh=_mesh,
    compiler_params=pltpu.CompilerParams(use_tc_tiling_on_sc=False),
    scratch_types=[
        pltpu.VMEM((2 * BPW, HALF), jnp.int32),
        pltpu.VMEM((4, HIST, EMBED), jnp.float32),
        pltpu.VMEM((BPW, EMBED), jnp.float32),
        pltpu.SemaphoreType.DMA,
        pltpu.SemaphoreType.DMA,
        pltpu.SemaphoreType.DMA,
        pltpu.SemaphoreType.DMA,
    ],
)
def _embed_pool(x_hbm, table_hbm, dummy_hbm, out_hbm, idx_v, rows_v, out_v,
                sem0, sem1, sem2, sem3):
    sems = (sem0, sem1, sem2, sem3)
    NBUF = 4
    wid = lax.axis_index("s") * NC + lax.axis_index("c")
    base = wid * BPW

    pltpu.sync_copy(x_hbm.at[pl.ds(2 * base, 2 * BPW)], idx_v)

    def issue(b, buf):
        pltpu.async_copy(table_hbm.at[idx_v.at[2 * b]],
                         rows_v.at[buf, pl.ds(0, HALF)], sems[buf])
        pltpu.async_copy(table_hbm.at[idx_v.at[2 * b + 1]],
                         rows_v.at[buf, pl.ds(HALF, HALF)], sems[buf])

    def wait(buf):
        pltpu.make_async_copy(dummy_hbm, rows_v.at[buf], sems[buf]).wait()

    def reduce(b, buf):
        U = 8

        def body(t, acc):
            acc = list(acc)
            for u in range(U):
                c = NVEC * (u % 2)
                for i in range(NVEC):
                    acc[c + i] = acc[c + i] + rows_v[
                        buf, t * U + u, pl.ds(LANES * i, LANES)]
            return tuple(acc)

        acc = lax.fori_loop(
            0, HIST // U, body,
            tuple(jnp.zeros((LANES,), jnp.float32) for _ in range(2 * NVEC)))
        for i in range(NVEC):
            out_v[b, pl.ds(LANES * i, LANES)] = (
                (acc[i] + acc[NVEC + i]) * (1.0 / HIST))

    for b in range(NBUF - 1):
        issue(b, b)

    def outer(j, _):
        b0 = j * NBUF
        for u in range(NBUF):
            nb = b0 + u + NBUF - 1

            @pl.when(nb < BPW)
            def _():
                issue(nb, (u + NBUF - 1) % NBUF)

            wait(u)
            reduce(b0 + u, u)
        return 0

    lax.fori_loop(0, BPW // NBUF, outer, 0)
    pltpu.sync_copy(out_v, out_hbm.at[pl.ds(base, BPW)])


def kernel(x, table):
    x2 = x.astype(jnp.int32).reshape(2 * BATCH, HALF)
    dummy = jnp.zeros((HIST, EMBED), jnp.float32)
    return _embed_pool(x2, table, dummy)
